# Initial kernel scaffold; baseline (speedup 1.0000x reference)
import jax
import jax.numpy as jnp
from jax.experimental import pallas as pl


def kernel(text_feature_general, text_feature_left, text_feature_mid_left, text_feature_mid_right, text_feature_right, text_length, radar_point_feat, radar_point_mask, weights):
    raise NotImplementedError("write your pallas kernel here")


# speedup vs baseline: 3.1116x; 3.1116x over previous
import functools
import math

import jax
import jax.numpy as jnp
from jax.experimental import pallas as pl
from jax.experimental.pallas import tpu as pltpu

B = 8
L = 512
T = 768
H = 128
P = 256
PD = 64
NB = 5
CH = 64
NCH = L // CH


def _mm(a, b):
    return jax.lax.dot_general(a, b, (((1,), (0,)), ((), ())),
                               preferred_element_type=jnp.float32)


def _lstm_mega_kernel(e0, e1, e2, e3, e4,
                      w1t, b1, w2t, b2, wih0t, bias0, whh0t, wcat1, bias1,
                      wlt, bl, lens,
                      out_ref,
                      xg, h0s, c0s, h1s, c1s, hls):
    k = pl.program_id(0)

    @pl.when(k == 0)
    def _init():
        z = jnp.zeros((NB, B, H), jnp.float32)
        h0s[...] = z
        c0s[...] = z
        h1s[...] = z
        c1s[...] = z
        hls[...] = z

    embs = (e0, e1, e2, e3, e4)

    for i in range(NB):
        x = embs[i][...].reshape(B * CH, T)
        h = _mm(x, w1t[i]) + b1[i]
        h = jnp.where(h >= 0, h, 0.01 * h)
        h = _mm(h, w2t[i]) + b2[i]
        xg[i] = (_mm(h, wih0t[i]) + bias0[i]).reshape(B, CH, 4 * H)

    whh0_v = [whh0t[i] for i in range(NB)]
    wcat1_v = [wcat1[i] for i in range(NB)]
    bias1_v = [bias1[i] for i in range(NB)]
    lens_v = [lens[i] for i in range(NB)]

    def gates(g):
        ig = jax.nn.sigmoid(g[:, 0 * H:1 * H])
        fg = jax.nn.sigmoid(g[:, 1 * H:2 * H])
        gg = jnp.tanh(g[:, 2 * H:3 * H])
        og = jax.nn.sigmoid(g[:, 3 * H:4 * H])
        return ig, fg, gg, og

    def step(t, carry):
        h0, c0, h1, c1, hl = carry
        t1 = k * CH + t + 1
        nh0, nc0, nh1, nc1, nhl = [], [], [], [], []
        for i in range(NB):
            xgt = xg[i, :, pl.ds(t, 1), :].reshape(B, 4 * H)
            g0 = xgt + _mm(h0[i], whh0_v[i])
            ig, fg, gg, og = gates(g0)
            c0n = fg * c0[i] + ig * gg
            h0n = og * jnp.tanh(c0n)
            g1 = _mm(jnp.concatenate([h0n, h1[i]], axis=1), wcat1_v[i]) + bias1_v[i]
            ig, fg, gg, og = gates(g1)
            c1n = fg * c1[i] + ig * gg
            h1n = og * jnp.tanh(c1n)
            m = lens_v[i] == t1
            nh0.append(h0n)
            nc0.append(c0n)
            nh1.append(h1n)
            nc1.append(c1n)
            nhl.append(jnp.where(m, h1n, hl[i]))
        return tuple(nh0), tuple(nc0), tuple(nh1), tuple(nc1), tuple(nhl)

    carry = (tuple(h0s[i] for i in range(NB)),
             tuple(c0s[i] for i in range(NB)),
             tuple(h1s[i] for i in range(NB)),
             tuple(c1s[i] for i in range(NB)),
             tuple(hls[i] for i in range(NB)))
    h0, c0, h1, c1, hl = jax.lax.fori_loop(0, CH, step, carry)

    for i in range(NB):
        h0s[i] = h0[i]
        c0s[i] = c0[i]
        h1s[i] = h1[i]
        c1s[i] = c1[i]
        hls[i] = hl[i]

    @pl.when(k == NCH - 1)
    def _fin():
        for i in range(NB):
            out_ref[i] = _mm(hl[i], wlt[i]) + bl[i]


def _attn_cls_kernel(feats, pts, rpm,
                     wqt, bq, wkt, bk, wvt, bv, wot, bo,
                     wc1t, bc1, wc2t, bc2, wc3t, bc3,
                     out1, out2, out3, out4, outc):
    pts2 = pts[...].reshape(B * P, PD)
    rpm_v = rpm[...]
    outs = (out1, out2, out3, out4)
    inv_sqrt = 1.0 / math.sqrt(H // 4)
    for j in range(4):
        kj = (_mm(pts2, wkt[j]) + bk[j]).reshape(B, P, H)
        vj = (_mm(pts2, wvt[j]) + bv[j]).reshape(B, P, H)
        fj = feats[j + 1]
        qj = _mm(fj, wqt[j]) + bq[j]
        maskj = rpm_v == (j + 1)
        heads = []
        for hh in range(4):
            sl = slice(32 * hh, 32 * hh + 32)
            lg = jnp.sum(kj[:, :, sl] * qj[:, None, sl], axis=-1) * inv_sqrt
            lg = jnp.where(maskj, lg, -1e30)
            lg = lg - jnp.max(lg, axis=1, keepdims=True)
            e = jnp.exp(lg)
            a = e / jnp.sum(e, axis=1, keepdims=True)
            heads.append(jnp.sum(a[:, :, None] * vj[:, :, sl], axis=1))
        o = jnp.concatenate(heads, axis=1)
        o = _mm(o, wot[j]) + bo[j]
        anyj = jnp.any(maskj, axis=1, keepdims=True)
        outs[j][...] = fj + jnp.where(anyj, o, 0.0)

    g = feats[0]
    h = jnp.maximum(_mm(g, wc1t[...]) + bc1[...], 0.0)
    h = jnp.maximum(_mm(h, wc2t[...]) + bc2[...], 0.0)
    outc[...] = _mm(h, wc3t[...]) + bc3[...]


def _full(shape):
    nd = len(shape)
    return pl.BlockSpec(shape, lambda k: (0,) * nd)


@jax.jit
def _run(tg, tl, tml, tmr, tr, text_length, rpf, rpm, weights):
    names = ('general', 'left', 'mid_left', 'mid_right', 'right')
    wb = [weights[n] for n in names]
    f32 = jnp.float32

    w1t = jnp.stack([w['W1'].T for w in wb])
    b1 = jnp.stack([w['b1'].reshape(1, H) for w in wb])
    w2t = jnp.stack([w['W2'].T for w in wb])
    b2 = jnp.stack([w['b2'].reshape(1, H) for w in wb])
    wih0t = jnp.stack([w['Wih0'].T for w in wb])
    bias0 = jnp.stack([(w['bih0'] + w['bhh0']).reshape(1, 4 * H) for w in wb])
    whh0t = jnp.stack([w['Whh0'].T for w in wb])
    wcat1 = jnp.stack([jnp.concatenate([w['Wih1'].T, w['Whh1'].T], axis=0)
                       for w in wb])
    bias1 = jnp.stack([(w['bih1'] + w['bhh1']).reshape(1, 4 * H) for w in wb])
    wlt = jnp.stack([w['Wl'].T for w in wb])
    bl = jnp.stack([w['bl'].reshape(1, H) for w in wb])
    lens = jnp.broadcast_to(text_length.T.astype(jnp.int32)[:, :, None],
                            (NB, B, H))

    emb_spec = pl.BlockSpec((B, CH, T), lambda k: (0, k, 0))
    feats = pl.pallas_call(
        _lstm_mega_kernel,
        grid=(NCH,),
        in_specs=[emb_spec] * 5 + [
            _full((NB, T, H)), _full((NB, 1, H)), _full((NB, H, H)),
            _full((NB, 1, H)), _full((NB, H, 4 * H)), _full((NB, 1, 4 * H)),
            _full((NB, H, 4 * H)), _full((NB, 2 * H, 4 * H)),
            _full((NB, 1, 4 * H)), _full((NB, H, H)), _full((NB, 1, H)),
            _full((NB, B, H)),
        ],
        out_specs=_full((NB, B, H)),
        out_shape=jax.ShapeDtypeStruct((NB, B, H), f32),
        scratch_shapes=[
            pltpu.VMEM((NB, B, CH, 4 * H), f32),
            pltpu.VMEM((NB, B, H), f32),
            pltpu.VMEM((NB, B, H), f32),
            pltpu.VMEM((NB, B, H), f32),
            pltpu.VMEM((NB, B, H), f32),
            pltpu.VMEM((NB, B, H), f32),
        ],
    )(tg, tl, tml, tmr, tr, w1t, b1, w2t, b2, wih0t, bias0, whh0t, wcat1,
      bias1, wlt, bl, lens)

    wa = weights['attn']
    wqt = jnp.stack([w['Wq'].T for w in wa])
    bq = jnp.stack([w['bq'].reshape(1, H) for w in wa])
    wkt = jnp.stack([w['Wk'].T for w in wa])
    bk = jnp.stack([w['bk'].reshape(1, H) for w in wa])
    wvt = jnp.stack([w['Wv'].T for w in wa])
    bv = jnp.stack([w['bv'].reshape(1, H) for w in wa])
    wot = jnp.stack([w['Wo'].T for w in wa])
    bo = jnp.stack([w['bo'].reshape(1, H) for w in wa])

    wc = weights['cls']
    wc1t = wc['W1'].T
    bc1 = wc['b1'].reshape(1, 128)
    wc2t = jnp.zeros((128, 128), f32).at[:, :16].set(wc['W2'].T)
    bc2 = jnp.zeros((1, 128), f32).at[0, :16].set(wc['b2'])
    wc3t = jnp.zeros((128, 128), f32).at[:16, :3].set(wc['W3'].T)
    bc3 = jnp.zeros((1, 128), f32).at[0, :3].set(wc['b3'])

    o1, o2, o3, o4, oc = pl.pallas_call(
        _attn_cls_kernel,
        grid=(1,),
        in_specs=[
            _full((NB, B, H)), _full((B, P, PD)), _full((B, P)),
            _full((4, H, H)), _full((4, 1, H)), _full((4, PD, H)),
            _full((4, 1, H)), _full((4, PD, H)), _full((4, 1, H)),
            _full((4, H, H)), _full((4, 1, H)),
            _full((H, H)), _full((1, H)), _full((H, H)), _full((1, H)),
            _full((H, H)), _full((1, H)),
        ],
        out_specs=[_full((B, H))] * 4 + [_full((B, H))],
        out_shape=[jax.ShapeDtypeStruct((B, H), f32)] * 4 +
                  [jax.ShapeDtypeStruct((B, H), f32)],
    )(feats, rpf, rpm, wqt, bq, wkt, bk, wvt, bv, wot, bo,
      wc1t, bc1, wc2t, bc2, wc3t, bc3)

    general = feats[0]
    return (general, (o1, o2, o3, o4), oc[:, :3], general)


def kernel(text_feature_general, text_feature_left, text_feature_mid_left,
           text_feature_mid_right, text_feature_right, text_length,
           radar_point_feat, radar_point_mask, weights):
    return _run(text_feature_general, text_feature_left, text_feature_mid_left,
                text_feature_mid_right, text_feature_right, text_length,
                radar_point_feat, radar_point_mask, weights)


# speedup vs baseline: 3.4067x; 1.0948x over previous
import functools
import math

import jax
import jax.numpy as jnp
from jax.experimental import pallas as pl
from jax.experimental.pallas import tpu as pltpu

B = 8
L = 512
T = 768
H = 128
P = 256
PD = 64
NB = 5
CH = 64
NCH = L // CH


def _mm(a, b):
    return jax.lax.dot_general(a, b, (((1,), (0,)), ((), ())),
                               preferred_element_type=jnp.float32)


def _mmb(a, b):
    return jax.lax.dot_general(a.astype(jnp.bfloat16), b,
                               (((1,), (0,)), ((), ())),
                               preferred_element_type=jnp.float32)


def _lstm_mega_kernel(e0, e1, e2, e3, e4,
                      w1t, b1, w2t, b2, wih0t, bias0, whh0t, wcat1, bias1,
                      wlt, bl, lens,
                      out_ref,
                      xg, h0s, c0s, h1s, c1s, hls):
    k = pl.program_id(0)

    @pl.when(k == 0)
    def _init():
        z = jnp.zeros((NB, B, H), jnp.float32)
        h0s[...] = z
        c0s[...] = z
        h1s[...] = z
        c1s[...] = z
        hls[...] = z

    embs = (e0, e1, e2, e3, e4)

    for i in range(NB):
        x = embs[i][...].reshape(B * CH, T)
        h = _mmb(x, w1t[i]) + b1[i]
        h = jnp.where(h >= 0, h, 0.01 * h)
        h = _mmb(h, w2t[i]) + b2[i]
        ht = jnp.swapaxes(h.reshape(B, CH, H), 0, 1).reshape(CH * B, H)
        xg[i] = _mmb(ht, wih0t[i]) + bias0[i]

    whh0_v = [whh0t[i] for i in range(NB)]
    wcat1_v = [wcat1[i] for i in range(NB)]
    bias1_v = [bias1[i] for i in range(NB)]
    lens_v = [lens[i] for i in range(NB)]

    def gates(g):
        ig = jax.nn.sigmoid(g[:, 0 * H:1 * H])
        fg = jax.nn.sigmoid(g[:, 1 * H:2 * H])
        gg = jnp.tanh(g[:, 2 * H:3 * H])
        og = jax.nn.sigmoid(g[:, 3 * H:4 * H])
        return ig, fg, gg, og

    def step(t, carry):
        h0, c0, h1, c1, hl = carry
        t1 = k * CH + t + 1
        nh0, nc0, nh1, nc1, nhl = [], [], [], [], []
        for i in range(NB):
            xgt = xg[i, pl.ds(t * B, B), :]
            g0 = xgt + _mmb(h0[i], whh0_v[i])
            ig, fg, gg, og = gates(g0)
            c0n = fg * c0[i] + ig * gg
            h0n = og * jnp.tanh(c0n)
            g1 = _mmb(jnp.concatenate([h0n, h1[i]], axis=1), wcat1_v[i]) + bias1_v[i]
            ig, fg, gg, og = gates(g1)
            c1n = fg * c1[i] + ig * gg
            h1n = og * jnp.tanh(c1n)
            m = lens_v[i] == t1
            nh0.append(h0n)
            nc0.append(c0n)
            nh1.append(h1n)
            nc1.append(c1n)
            nhl.append(jnp.where(m, h1n, hl[i]))
        return tuple(nh0), tuple(nc0), tuple(nh1), tuple(nc1), tuple(nhl)

    carry = (tuple(h0s[i] for i in range(NB)),
             tuple(c0s[i] for i in range(NB)),
             tuple(h1s[i] for i in range(NB)),
             tuple(c1s[i] for i in range(NB)),
             tuple(hls[i] for i in range(NB)))
    h0, c0, h1, c1, hl = jax.lax.fori_loop(0, CH, step, carry)

    for i in range(NB):
        h0s[i] = h0[i]
        c0s[i] = c0[i]
        h1s[i] = h1[i]
        c1s[i] = c1[i]
        hls[i] = hl[i]

    @pl.when(k == NCH - 1)
    def _fin():
        for i in range(NB):
            out_ref[i] = _mm(hl[i], wlt[i]) + bl[i]


def _attn_cls_kernel(feats, pts, rpm,
                     wqt, bq, wkt, bk, wvt, bv, wot, bo,
                     wc1t, bc1, wc2t, bc2, wc3t, bc3,
                     out1, out2, out3, out4, outc):
    pts2 = pts[...].reshape(B * P, PD)
    rpm_v = rpm[...]
    outs = (out1, out2, out3, out4)
    inv_sqrt = 1.0 / math.sqrt(H // 4)
    for j in range(4):
        kj = (_mm(pts2, wkt[j]) + bk[j]).reshape(B, P, H)
        vj = (_mm(pts2, wvt[j]) + bv[j]).reshape(B, P, H)
        fj = feats[j + 1]
        qj = _mm(fj, wqt[j]) + bq[j]
        maskj = rpm_v == (j + 1)
        heads = []
        for hh in range(4):
            sl = slice(32 * hh, 32 * hh + 32)
            lg = jnp.sum(kj[:, :, sl] * qj[:, None, sl], axis=-1) * inv_sqrt
            lg = jnp.where(maskj, lg, -1e30)
            lg = lg - jnp.max(lg, axis=1, keepdims=True)
            e = jnp.exp(lg)
            a = e / jnp.sum(e, axis=1, keepdims=True)
            heads.append(jnp.sum(a[:, :, None] * vj[:, :, sl], axis=1))
        o = jnp.concatenate(heads, axis=1)
        o = _mm(o, wot[j]) + bo[j]
        anyj = jnp.any(maskj, axis=1, keepdims=True)
        outs[j][...] = fj + jnp.where(anyj, o, 0.0)

    g = feats[0]
    h = jnp.maximum(_mm(g, wc1t[...]) + bc1[...], 0.0)
    h = jnp.maximum(_mm(h, wc2t[...]) + bc2[...], 0.0)
    outc[...] = _mm(h, wc3t[...]) + bc3[...]


def _full(shape):
    nd = len(shape)
    return pl.BlockSpec(shape, lambda k: (0,) * nd)


@jax.jit
def _run(tg, tl, tml, tmr, tr, text_length, rpf, rpm, weights):
    names = ('general', 'left', 'mid_left', 'mid_right', 'right')
    wb = [weights[n] for n in names]
    f32 = jnp.float32

    bf16 = jnp.bfloat16
    w1t = jnp.stack([w['W1'].T for w in wb]).astype(bf16)
    b1 = jnp.stack([w['b1'].reshape(1, H) for w in wb])
    w2t = jnp.stack([w['W2'].T for w in wb]).astype(bf16)
    b2 = jnp.stack([w['b2'].reshape(1, H) for w in wb])
    wih0t = jnp.stack([w['Wih0'].T for w in wb]).astype(bf16)
    bias0 = jnp.stack([(w['bih0'] + w['bhh0']).reshape(1, 4 * H) for w in wb])
    whh0t = jnp.stack([w['Whh0'].T for w in wb]).astype(bf16)
    wcat1 = jnp.stack([jnp.concatenate([w['Wih1'].T, w['Whh1'].T], axis=0)
                       for w in wb]).astype(bf16)
    bias1 = jnp.stack([(w['bih1'] + w['bhh1']).reshape(1, 4 * H) for w in wb])
    wlt = jnp.stack([w['Wl'].T for w in wb])
    bl = jnp.stack([w['bl'].reshape(1, H) for w in wb])
    lens = jnp.broadcast_to(text_length.T.astype(jnp.int32)[:, :, None],
                            (NB, B, H))

    emb_spec = pl.BlockSpec((B, CH, T), lambda k: (0, k, 0))
    feats = pl.pallas_call(
        _lstm_mega_kernel,
        grid=(NCH,),
        in_specs=[emb_spec] * 5 + [
            _full((NB, T, H)), _full((NB, 1, H)), _full((NB, H, H)),
            _full((NB, 1, H)), _full((NB, H, 4 * H)), _full((NB, 1, 4 * H)),
            _full((NB, H, 4 * H)), _full((NB, 2 * H, 4 * H)),
            _full((NB, 1, 4 * H)), _full((NB, H, H)), _full((NB, 1, H)),
            _full((NB, B, H)),
        ],
        out_specs=_full((NB, B, H)),
        out_shape=jax.ShapeDtypeStruct((NB, B, H), f32),
        scratch_shapes=[
            pltpu.VMEM((NB, CH * B, 4 * H), f32),
            pltpu.VMEM((NB, B, H), f32),
            pltpu.VMEM((NB, B, H), f32),
            pltpu.VMEM((NB, B, H), f32),
            pltpu.VMEM((NB, B, H), f32),
            pltpu.VMEM((NB, B, H), f32),
        ],
    )(tg, tl, tml, tmr, tr, w1t, b1, w2t, b2, wih0t, bias0, whh0t, wcat1,
      bias1, wlt, bl, lens)

    wa = weights['attn']
    wqt = jnp.stack([w['Wq'].T for w in wa])
    bq = jnp.stack([w['bq'].reshape(1, H) for w in wa])
    wkt = jnp.stack([w['Wk'].T for w in wa])
    bk = jnp.stack([w['bk'].reshape(1, H) for w in wa])
    wvt = jnp.stack([w['Wv'].T for w in wa])
    bv = jnp.stack([w['bv'].reshape(1, H) for w in wa])
    wot = jnp.stack([w['Wo'].T for w in wa])
    bo = jnp.stack([w['bo'].reshape(1, H) for w in wa])

    wc = weights['cls']
    wc1t = wc['W1'].T
    bc1 = wc['b1'].reshape(1, 128)
    wc2t = jnp.zeros((128, 128), f32).at[:, :16].set(wc['W2'].T)
    bc2 = jnp.zeros((1, 128), f32).at[0, :16].set(wc['b2'])
    wc3t = jnp.zeros((128, 128), f32).at[:16, :3].set(wc['W3'].T)
    bc3 = jnp.zeros((1, 128), f32).at[0, :3].set(wc['b3'])

    o1, o2, o3, o4, oc = pl.pallas_call(
        _attn_cls_kernel,
        grid=(1,),
        in_specs=[
            _full((NB, B, H)), _full((B, P, PD)), _full((B, P)),
            _full((4, H, H)), _full((4, 1, H)), _full((4, PD, H)),
            _full((4, 1, H)), _full((4, PD, H)), _full((4, 1, H)),
            _full((4, H, H)), _full((4, 1, H)),
            _full((H, H)), _full((1, H)), _full((H, H)), _full((1, H)),
            _full((H, H)), _full((1, H)),
        ],
        out_specs=[_full((B, H))] * 4 + [_full((B, H))],
        out_shape=[jax.ShapeDtypeStruct((B, H), f32)] * 4 +
                  [jax.ShapeDtypeStruct((B, H), f32)],
    )(feats, rpf, rpm, wqt, bq, wkt, bk, wvt, bv, wot, bo,
      wc1t, bc1, wc2t, bc2, wc3t, bc3)

    general = feats[0]
    return (general, (o1, o2, o3, o4), oc[:, :3], general)


def kernel(text_feature_general, text_feature_left, text_feature_mid_left,
           text_feature_mid_right, text_feature_right, text_length,
           radar_point_feat, radar_point_mask, weights):
    return _run(text_feature_general, text_feature_left, text_feature_mid_left,
                text_feature_mid_right, text_feature_right, text_length,
                radar_point_feat, radar_point_mask, weights)


# speedup vs baseline: 3.8674x; 1.1352x over previous
import functools
import math

import jax
import jax.numpy as jnp
from jax.experimental import pallas as pl
from jax.experimental.pallas import tpu as pltpu

B = 8
L = 512
T = 768
H = 128
P = 256
PD = 64
NB = 5
CH = 64
NCH = L // CH


def _mm(a, b):
    return jax.lax.dot_general(a, b, (((1,), (0,)), ((), ())),
                               preferred_element_type=jnp.float32)


def _mmb(a, b):
    return jax.lax.dot_general(a.astype(jnp.bfloat16), b,
                               (((1,), (0,)), ((), ())),
                               preferred_element_type=jnp.float32)


def _lstm_mega_kernel(e0, e1, e2, e3, e4,
                      w1t, b1, w2t, b2, wih0t, bias0, whh0t, wcat1, bias1,
                      wlt, bl, lens,
                      out_ref,
                      xg, h0s, c0s, h1s, c1s, hls):
    k = pl.program_id(0)

    @pl.when(k == 0)
    def _init():
        z = jnp.zeros((NB, B, H), jnp.float32)
        h0s[...] = z
        c0s[...] = z
        h1s[...] = z
        c1s[...] = z
        hls[...] = z

    embs = (e0, e1, e2, e3, e4)

    for i in range(NB):
        x = embs[i][...].reshape(B * CH, T)
        h = _mmb(x, w1t[i]) + b1[i]
        h = jnp.where(h >= 0, h, 0.01 * h)
        h = _mmb(h, w2t[i]) + b2[i]
        ht = jnp.swapaxes(h.reshape(B, CH, H), 0, 1).reshape(CH * B, H)
        xg[i] = _mmb(ht, wih0t[i]) + bias0[i]

    def gates(g):
        ig = jax.nn.sigmoid(g[:, 0 * H:1 * H])
        fg = jax.nn.sigmoid(g[:, 1 * H:2 * H])
        gg = jnp.tanh(g[:, 2 * H:3 * H])
        og = jax.nn.sigmoid(g[:, 3 * H:4 * H])
        return ig, fg, gg, og

    def substep(t, carry):
        h0, c0, h1, c1, hl = carry
        t1 = k * CH + t + 1
        nh0, nc0, nh1, nc1, nhl = [], [], [], [], []
        for i in range(NB):
            xgt = xg[i, pl.ds(t * B, B), :]
            g0 = xgt + _mmb(h0[i], whh0t[i])
            ig, fg, gg, og = gates(g0)
            c0n = fg * c0[i] + ig * gg
            h0n = og * jnp.tanh(c0n)
            g1 = _mmb(jnp.concatenate([h0n, h1[i]], axis=1), wcat1[i]) + bias1[i]
            ig, fg, gg, og = gates(g1)
            c1n = fg * c1[i] + ig * gg
            h1n = og * jnp.tanh(c1n)
            m = lens[i] == t1
            nh0.append(h0n)
            nc0.append(c0n)
            nh1.append(h1n)
            nc1.append(c1n)
            nhl.append(jnp.where(m, h1n, hl[i]))
        return tuple(nh0), tuple(nc0), tuple(nh1), tuple(nc1), tuple(nhl)

    UNROLL = 4

    def step(u, carry):
        for r in range(UNROLL):
            carry = substep(u * UNROLL + r, carry)
        return carry

    carry = (tuple(h0s[i] for i in range(NB)),
             tuple(c0s[i] for i in range(NB)),
             tuple(h1s[i] for i in range(NB)),
             tuple(c1s[i] for i in range(NB)),
             tuple(hls[i] for i in range(NB)))
    h0, c0, h1, c1, hl = jax.lax.fori_loop(0, CH // UNROLL, step, carry)

    for i in range(NB):
        h0s[i] = h0[i]
        c0s[i] = c0[i]
        h1s[i] = h1[i]
        c1s[i] = c1[i]
        hls[i] = hl[i]

    @pl.when(k == NCH - 1)
    def _fin():
        for i in range(NB):
            out_ref[i] = _mm(hl[i], wlt[i]) + bl[i]


def _attn_cls_kernel(feats, pts, rpm,
                     wqt, bq, wkt, bk, wvt, bv, wot, bo,
                     wc1t, bc1, wc2t, bc2, wc3t, bc3,
                     out1, out2, out3, out4, outc):
    pts2 = pts[...].reshape(B * P, PD)
    rpm_v = rpm[...]
    outs = (out1, out2, out3, out4)
    inv_sqrt = 1.0 / math.sqrt(H // 4)
    for j in range(4):
        kj = (_mm(pts2, wkt[j]) + bk[j]).reshape(B, P, H)
        vj = (_mm(pts2, wvt[j]) + bv[j]).reshape(B, P, H)
        fj = feats[j + 1]
        qj = _mm(fj, wqt[j]) + bq[j]
        maskj = rpm_v == (j + 1)
        heads = []
        for hh in range(4):
            sl = slice(32 * hh, 32 * hh + 32)
            lg = jnp.sum(kj[:, :, sl] * qj[:, None, sl], axis=-1) * inv_sqrt
            lg = jnp.where(maskj, lg, -1e30)
            lg = lg - jnp.max(lg, axis=1, keepdims=True)
            e = jnp.exp(lg)
            a = e / jnp.sum(e, axis=1, keepdims=True)
            heads.append(jnp.sum(a[:, :, None] * vj[:, :, sl], axis=1))
        o = jnp.concatenate(heads, axis=1)
        o = _mm(o, wot[j]) + bo[j]
        anyj = jnp.any(maskj, axis=1, keepdims=True)
        outs[j][...] = fj + jnp.where(anyj, o, 0.0)

    g = feats[0]
    h = jnp.maximum(_mm(g, wc1t[...]) + bc1[...], 0.0)
    h = jnp.maximum(_mm(h, wc2t[...]) + bc2[...], 0.0)
    outc[...] = _mm(h, wc3t[...]) + bc3[...]


def _full(shape):
    nd = len(shape)
    return pl.BlockSpec(shape, lambda k: (0,) * nd)


@jax.jit
def _run(tg, tl, tml, tmr, tr, text_length, rpf, rpm, weights):
    names = ('general', 'left', 'mid_left', 'mid_right', 'right')
    wb = [weights[n] for n in names]
    f32 = jnp.float32

    bf16 = jnp.bfloat16
    w1t = jnp.stack([w['W1'].T for w in wb]).astype(bf16)
    b1 = jnp.stack([w['b1'].reshape(1, H) for w in wb])
    w2t = jnp.stack([w['W2'].T for w in wb]).astype(bf16)
    b2 = jnp.stack([w['b2'].reshape(1, H) for w in wb])
    wih0t = jnp.stack([w['Wih0'].T for w in wb]).astype(bf16)
    bias0 = jnp.stack([(w['bih0'] + w['bhh0']).reshape(1, 4 * H) for w in wb])
    whh0t = jnp.stack([w['Whh0'].T for w in wb]).astype(bf16)
    wcat1 = jnp.stack([jnp.concatenate([w['Wih1'].T, w['Whh1'].T], axis=0)
                       for w in wb]).astype(bf16)
    bias1 = jnp.stack([(w['bih1'] + w['bhh1']).reshape(1, 4 * H) for w in wb])
    wlt = jnp.stack([w['Wl'].T for w in wb])
    bl = jnp.stack([w['bl'].reshape(1, H) for w in wb])
    lens = jnp.broadcast_to(text_length.T.astype(jnp.int32)[:, :, None],
                            (NB, B, H))

    emb_spec = pl.BlockSpec((B, CH, T), lambda k: (0, k, 0))
    feats = pl.pallas_call(
        _lstm_mega_kernel,
        grid=(NCH,),
        in_specs=[emb_spec] * 5 + [
            _full((NB, T, H)), _full((NB, 1, H)), _full((NB, H, H)),
            _full((NB, 1, H)), _full((NB, H, 4 * H)), _full((NB, 1, 4 * H)),
            _full((NB, H, 4 * H)), _full((NB, 2 * H, 4 * H)),
            _full((NB, 1, 4 * H)), _full((NB, H, H)), _full((NB, 1, H)),
            _full((NB, B, H)),
        ],
        out_specs=_full((NB, B, H)),
        out_shape=jax.ShapeDtypeStruct((NB, B, H), f32),
        scratch_shapes=[
            pltpu.VMEM((NB, CH * B, 4 * H), f32),
            pltpu.VMEM((NB, B, H), f32),
            pltpu.VMEM((NB, B, H), f32),
            pltpu.VMEM((NB, B, H), f32),
            pltpu.VMEM((NB, B, H), f32),
            pltpu.VMEM((NB, B, H), f32),
        ],
    )(tg, tl, tml, tmr, tr, w1t, b1, w2t, b2, wih0t, bias0, whh0t, wcat1,
      bias1, wlt, bl, lens)

    wa = weights['attn']
    wqt = jnp.stack([w['Wq'].T for w in wa])
    bq = jnp.stack([w['bq'].reshape(1, H) for w in wa])
    wkt = jnp.stack([w['Wk'].T for w in wa])
    bk = jnp.stack([w['bk'].reshape(1, H) for w in wa])
    wvt = jnp.stack([w['Wv'].T for w in wa])
    bv = jnp.stack([w['bv'].reshape(1, H) for w in wa])
    wot = jnp.stack([w['Wo'].T for w in wa])
    bo = jnp.stack([w['bo'].reshape(1, H) for w in wa])

    wc = weights['cls']
    wc1t = wc['W1'].T
    bc1 = wc['b1'].reshape(1, 128)
    wc2t = jnp.zeros((128, 128), f32).at[:, :16].set(wc['W2'].T)
    bc2 = jnp.zeros((1, 128), f32).at[0, :16].set(wc['b2'])
    wc3t = jnp.zeros((128, 128), f32).at[:16, :3].set(wc['W3'].T)
    bc3 = jnp.zeros((1, 128), f32).at[0, :3].set(wc['b3'])

    o1, o2, o3, o4, oc = pl.pallas_call(
        _attn_cls_kernel,
        grid=(1,),
        in_specs=[
            _full((NB, B, H)), _full((B, P, PD)), _full((B, P)),
            _full((4, H, H)), _full((4, 1, H)), _full((4, PD, H)),
            _full((4, 1, H)), _full((4, PD, H)), _full((4, 1, H)),
            _full((4, H, H)), _full((4, 1, H)),
            _full((H, H)), _full((1, H)), _full((H, H)), _full((1, H)),
            _full((H, H)), _full((1, H)),
        ],
        out_specs=[_full((B, H))] * 4 + [_full((B, H))],
        out_shape=[jax.ShapeDtypeStruct((B, H), f32)] * 4 +
                  [jax.ShapeDtypeStruct((B, H), f32)],
    )(feats, rpf, rpm, wqt, bq, wkt, bk, wvt, bv, wot, bo,
      wc1t, bc1, wc2t, bc2, wc3t, bc3)

    general = feats[0]
    return (general, (o1, o2, o3, o4), oc[:, :3], general)


def kernel(text_feature_general, text_feature_left, text_feature_mid_left,
           text_feature_mid_right, text_feature_right, text_length,
           radar_point_feat, radar_point_mask, weights):
    return _run(text_feature_general, text_feature_left, text_feature_mid_left,
                text_feature_mid_right, text_feature_right, text_length,
                radar_point_feat, radar_point_mask, weights)


# speedup vs baseline: 6.8463x; 1.7703x over previous
import functools
import math

import jax
import jax.numpy as jnp
from jax.experimental import pallas as pl
from jax.experimental.pallas import tpu as pltpu

B = 8
L = 512
T = 768
H = 128
P = 256
PD = 64
NB = 5
CH = 64
NCH = L // CH


def _mm(a, b):
    return jax.lax.dot_general(a, b, (((1,), (0,)), ((), ())),
                               preferred_element_type=jnp.float32)


def _mmb(a, b):
    return jax.lax.dot_general(a.astype(jnp.bfloat16), b,
                               (((1,), (0,)), ((), ())),
                               preferred_element_type=jnp.float32)


def _lstm_mega_kernel(e0, e1, e2, e3, e4,
                      w1t, b1, w2t, b2, wih0t, bias0, whh0t, wih1t, whh1t,
                      bias1, wlt, bl, lens,
                      out_ref,
                      xg, hseq, h0s, c0s, h1s, c1s, hls):
    k = pl.program_id(0)

    @pl.when(k == 0)
    def _init():
        z = jnp.zeros((NB, B, H), jnp.float32)
        h0s[...] = z
        c0s[...] = z
        h1s[...] = z
        c1s[...] = z
        hls[...] = z

    embs = (e0, e1, e2, e3, e4)

    for i in range(NB):
        x = embs[i][...].reshape(B * CH, T)
        h = _mmb(x, w1t[i]) + b1[i]
        h = jnp.where(h >= 0, h, 0.01 * h)
        h = _mmb(h, w2t[i]) + b2[i]
        ht = jnp.swapaxes(h.reshape(B, CH, H), 0, 1).reshape(CH * B, H)
        xg[i] = _mmb(ht, wih0t[i]) + bias0[i]

    def gates(g):
        ig = jax.nn.sigmoid(g[:, 0 * H:1 * H])
        fg = jax.nn.sigmoid(g[:, 1 * H:2 * H])
        gg = jnp.tanh(g[:, 2 * H:3 * H])
        og = jax.nn.sigmoid(g[:, 3 * H:4 * H])
        return ig, fg, gg, og

    UNROLL = 4

    def substep0(t, carry):
        h0, c0 = carry
        nh0, nc0 = [], []
        for i in range(NB):
            g0 = xg[i, pl.ds(t * B, B), :] + _mmb(h0[i], whh0t[i])
            ig, fg, gg, og = gates(g0)
            c0n = fg * c0[i] + ig * gg
            h0n = og * jnp.tanh(c0n)
            hseq[i, pl.ds(t * B, B), :] = h0n
            nh0.append(h0n)
            nc0.append(c0n)
        return tuple(nh0), tuple(nc0)

    def step0(u, carry):
        for r in range(UNROLL):
            carry = substep0(u * UNROLL + r, carry)
        return carry

    carry = (tuple(h0s[i] for i in range(NB)),
             tuple(c0s[i] for i in range(NB)))
    h0, c0 = jax.lax.fori_loop(0, CH // UNROLL, step0, carry)
    for i in range(NB):
        h0s[i] = h0[i]
        c0s[i] = c0[i]

    for i in range(NB):
        xg[i] = _mmb(hseq[i], wih1t[i]) + bias1[i]

    def substep1(t, carry):
        h1, c1, hl = carry
        t1 = k * CH + t + 1
        nh1, nc1, nhl = [], [], []
        for i in range(NB):
            g1 = xg[i, pl.ds(t * B, B), :] + _mmb(h1[i], whh1t[i])
            ig, fg, gg, og = gates(g1)
            c1n = fg * c1[i] + ig * gg
            h1n = og * jnp.tanh(c1n)
            m = lens[i] == t1
            nh1.append(h1n)
            nc1.append(c1n)
            nhl.append(jnp.where(m, h1n, hl[i]))
        return tuple(nh1), tuple(nc1), tuple(nhl)

    def step1(u, carry):
        for r in range(UNROLL):
            carry = substep1(u * UNROLL + r, carry)
        return carry

    carry = (tuple(h1s[i] for i in range(NB)),
             tuple(c1s[i] for i in range(NB)),
             tuple(hls[i] for i in range(NB)))
    h1, c1, hl = jax.lax.fori_loop(0, CH // UNROLL, step1, carry)
    for i in range(NB):
        h1s[i] = h1[i]
        c1s[i] = c1[i]
        hls[i] = hl[i]

    @pl.when(k == NCH - 1)
    def _fin():
        for i in range(NB):
            out_ref[i] = _mm(hl[i], wlt[i]) + bl[i]


def _attn_cls_kernel(feats, pts, rpm,
                     wqt, bq, wkt, bk, wvt, bv, wot, bo,
                     wc1t, bc1, wc2t, bc2, wc3t, bc3,
                     out1, out2, out3, out4, outc):
    pts2 = pts[...].reshape(B * P, PD)
    rpm_v = rpm[...]
    outs = (out1, out2, out3, out4)
    inv_sqrt = 1.0 / math.sqrt(H // 4)
    for j in range(4):
        kj = (_mm(pts2, wkt[j]) + bk[j]).reshape(B, P, H)
        vj = (_mm(pts2, wvt[j]) + bv[j]).reshape(B, P, H)
        fj = feats[j + 1]
        qj = _mm(fj, wqt[j]) + bq[j]
        maskj = rpm_v == (j + 1)
        heads = []
        for hh in range(4):
            sl = slice(32 * hh, 32 * hh + 32)
            lg = jnp.sum(kj[:, :, sl] * qj[:, None, sl], axis=-1) * inv_sqrt
            lg = jnp.where(maskj, lg, -1e30)
            lg = lg - jnp.max(lg, axis=1, keepdims=True)
            e = jnp.exp(lg)
            a = e / jnp.sum(e, axis=1, keepdims=True)
            heads.append(jnp.sum(a[:, :, None] * vj[:, :, sl], axis=1))
        o = jnp.concatenate(heads, axis=1)
        o = _mm(o, wot[j]) + bo[j]
        anyj = jnp.any(maskj, axis=1, keepdims=True)
        outs[j][...] = fj + jnp.where(anyj, o, 0.0)

    g = feats[0]
    h = jnp.maximum(_mm(g, wc1t[...]) + bc1[...], 0.0)
    h = jnp.maximum(_mm(h, wc2t[...]) + bc2[...], 0.0)
    outc[...] = _mm(h, wc3t[...]) + bc3[...]


def _full(shape):
    nd = len(shape)
    return pl.BlockSpec(shape, lambda k: (0,) * nd)


@jax.jit
def _run(tg, tl, tml, tmr, tr, text_length, rpf, rpm, weights):
    names = ('general', 'left', 'mid_left', 'mid_right', 'right')
    wb = [weights[n] for n in names]
    f32 = jnp.float32

    bf16 = jnp.bfloat16
    w1t = jnp.stack([w['W1'].T for w in wb]).astype(bf16)
    b1 = jnp.stack([w['b1'].reshape(1, H) for w in wb])
    w2t = jnp.stack([w['W2'].T for w in wb]).astype(bf16)
    b2 = jnp.stack([w['b2'].reshape(1, H) for w in wb])
    wih0t = jnp.stack([w['Wih0'].T for w in wb]).astype(bf16)
    bias0 = jnp.stack([(w['bih0'] + w['bhh0']).reshape(1, 4 * H) for w in wb])
    whh0t = jnp.stack([w['Whh0'].T for w in wb]).astype(bf16)
    wih1t = jnp.stack([w['Wih1'].T for w in wb]).astype(bf16)
    whh1t = jnp.stack([w['Whh1'].T for w in wb]).astype(bf16)
    bias1 = jnp.stack([(w['bih1'] + w['bhh1']).reshape(1, 4 * H) for w in wb])
    wlt = jnp.stack([w['Wl'].T for w in wb])
    bl = jnp.stack([w['bl'].reshape(1, H) for w in wb])
    lens = jnp.broadcast_to(text_length.T.astype(jnp.int32)[:, :, None],
                            (NB, B, H))

    emb_spec = pl.BlockSpec((B, CH, T), lambda k: (0, k, 0))
    feats = pl.pallas_call(
        _lstm_mega_kernel,
        grid=(NCH,),
        in_specs=[emb_spec] * 5 + [
            _full((NB, T, H)), _full((NB, 1, H)), _full((NB, H, H)),
            _full((NB, 1, H)), _full((NB, H, 4 * H)), _full((NB, 1, 4 * H)),
            _full((NB, H, 4 * H)), _full((NB, H, 4 * H)),
            _full((NB, H, 4 * H)),
            _full((NB, 1, 4 * H)), _full((NB, H, H)), _full((NB, 1, H)),
            _full((NB, B, H)),
        ],
        out_specs=_full((NB, B, H)),
        out_shape=jax.ShapeDtypeStruct((NB, B, H), f32),
        scratch_shapes=[
            pltpu.VMEM((NB, CH * B, 4 * H), f32),
            pltpu.VMEM((NB, CH * B, H), f32),
            pltpu.VMEM((NB, B, H), f32),
            pltpu.VMEM((NB, B, H), f32),
            pltpu.VMEM((NB, B, H), f32),
            pltpu.VMEM((NB, B, H), f32),
            pltpu.VMEM((NB, B, H), f32),
        ],
    )(tg, tl, tml, tmr, tr, w1t, b1, w2t, b2, wih0t, bias0, whh0t, wih1t,
      whh1t, bias1, wlt, bl, lens)

    wa = weights['attn']
    wqt = jnp.stack([w['Wq'].T for w in wa])
    bq = jnp.stack([w['bq'].reshape(1, H) for w in wa])
    wkt = jnp.stack([w['Wk'].T for w in wa])
    bk = jnp.stack([w['bk'].reshape(1, H) for w in wa])
    wvt = jnp.stack([w['Wv'].T for w in wa])
    bv = jnp.stack([w['bv'].reshape(1, H) for w in wa])
    wot = jnp.stack([w['Wo'].T for w in wa])
    bo = jnp.stack([w['bo'].reshape(1, H) for w in wa])

    wc = weights['cls']
    wc1t = wc['W1'].T
    bc1 = wc['b1'].reshape(1, 128)
    wc2t = jnp.zeros((128, 128), f32).at[:, :16].set(wc['W2'].T)
    bc2 = jnp.zeros((1, 128), f32).at[0, :16].set(wc['b2'])
    wc3t = jnp.zeros((128, 128), f32).at[:16, :3].set(wc['W3'].T)
    bc3 = jnp.zeros((1, 128), f32).at[0, :3].set(wc['b3'])

    o1, o2, o3, o4, oc = pl.pallas_call(
        _attn_cls_kernel,
        grid=(1,),
        in_specs=[
            _full((NB, B, H)), _full((B, P, PD)), _full((B, P)),
            _full((4, H, H)), _full((4, 1, H)), _full((4, PD, H)),
            _full((4, 1, H)), _full((4, PD, H)), _full((4, 1, H)),
            _full((4, H, H)), _full((4, 1, H)),
            _full((H, H)), _full((1, H)), _full((H, H)), _full((1, H)),
            _full((H, H)), _full((1, H)),
        ],
        out_specs=[_full((B, H))] * 4 + [_full((B, H))],
        out_shape=[jax.ShapeDtypeStruct((B, H), f32)] * 4 +
                  [jax.ShapeDtypeStruct((B, H), f32)],
    )(feats, rpf, rpm, wqt, bq, wkt, bk, wvt, bv, wot, bo,
      wc1t, bc1, wc2t, bc2, wc3t, bc3)

    general = feats[0]
    return (general, (o1, o2, o3, o4), oc[:, :3], general)


def kernel(text_feature_general, text_feature_left, text_feature_mid_left,
           text_feature_mid_right, text_feature_right, text_length,
           radar_point_feat, radar_point_mask, weights):
    return _run(text_feature_general, text_feature_left, text_feature_mid_left,
                text_feature_mid_right, text_feature_right, text_length,
                radar_point_feat, radar_point_mask, weights)
